# baseline (speedup 1.0000x reference)
import functools

import jax
import jax.numpy as jnp
from jax import lax
from jax.experimental import pallas as pl
from jax.experimental.pallas import tpu as pltpu
from jax.experimental.pallas import tpu_sc as plsc

N = 10000
E = 320000
D = 128
G = 64
NC, NS = 2, 16
NW = NC * NS
CHUNK = 128
EPW = 10240
E_PAD = NW * EPW
NCH = EPW // CHUNK
TOT_CH = NW * NCH
NP = 10240
NT = NP
PAD_DST = 10100
RB = 1024
GRID = NP // RB
BI = 256
CJ = 512
F32 = jnp.float32


def _sc_segment_sum(tbl_p, idx_packed, zeros, d2, npass):
    rpt = NT // NS
    spt = NP // NS
    mesh = plsc.VectorSubcoreMesh(core_axis_name="c", subcore_axis_name="s")

    @functools.partial(
        pl.kernel,
        out_type=jax.ShapeDtypeStruct((NC, npass, NT, d2), F32),
        mesh=mesh,
        scratch_types=[
            pltpu.VMEM((8, 2, CHUNK), jnp.int32),
            pltpu.VMEM((4, CHUNK, d2), F32),
            pltpu.VMEM_SHARED((NP, d2), F32),
            pltpu.VMEM_SHARED((NT, d2), F32),
            pltpu.SemaphoreType.DMA,
            pltpu.SemaphoreType.DMA,
            pltpu.SemaphoreType.DMA,
            pltpu.SemaphoreType.DMA,
            pltpu.SemaphoreType.DMA,
            pltpu.SemaphoreType.DMA,
        ],
        compiler_params=pltpu.CompilerParams(use_tc_tiling_on_sc=False),
    )
    def k(tbl_hbm, idx_hbm, zeros_hbm, out_hbm, idx_v, rows_v, tbl_sh,
          acc_sh, si0, si1, si2, si3, sr0, sr1):
        cid = lax.axis_index("c")
        sid = lax.axis_index("s")
        wid = sid * NC + cid
        base = wid * NCH
        sis = (si0, si1)
        srs = (si2, si3)
        sss = (sr0, sr1)

        def idx_start(c, s8, pi):
            pltpu.async_copy(idx_hbm.at[base + c], idx_v.at[s8], sis[pi])

        def idx_wait(c, s8, pi):
            pltpu.make_async_copy(idx_hbm.at[base + c], idx_v.at[s8],
                                  sis[pi]).wait()

        def row_start(s8, r, pi):
            pltpu.async_copy(tbl_sh.at[idx_v.at[s8, 0]], rows_v.at[r],
                             srs[pi])

        def row_wait(r, pi):
            pltpu.make_async_copy(tbl_sh.at[idx_v.at[0, 0]], rows_v.at[r],
                                  srs[pi]).wait()

        def sc_wait(pi):
            pltpu.make_async_copy(rows_v.at[0], acc_sh.at[idx_v.at[0, 1]],
                                  sss[pi]).wait()

        for p in range(npass):
            pltpu.sync_copy(tbl_hbm.at[p, pl.ds(sid * spt, spt)],
                            tbl_sh.at[pl.ds(sid * spt, spt)])
            pltpu.sync_copy(zeros_hbm.at[pl.ds(sid * rpt, rpt)],
                            acc_sh.at[pl.ds(sid * rpt, rpt)])
            plsc.subcore_barrier()
            idx_start(0, 0, 0)
            idx_start(1, 1, 1)
            idx_wait(0, 0, 0)
            row_start(0, 0, 0)
            idx_wait(1, 1, 1)
            row_start(1, 1, 1)
            idx_start(2, 2, 0)
            idx_start(3, 3, 1)

            def body(q, carry):
                for u in range(8):
                    j = q * 8 + u
                    pi = u % 2
                    r = u % 4
                    row_wait(r, pi)
                    if u >= 2:
                        sc_wait(pi)
                    else:
                        @pl.when(q > 0)
                        def _(pi=pi):
                            sc_wait(pi)
                    pltpu.async_copy(rows_v.at[r], acc_sh.at[idx_v.at[u, 1]],
                                     sss[pi], add=True)

                    @pl.when(j + 2 < NCH)
                    def _(j=j, u=u, r=r, pi=pi):
                        idx_wait(j + 2, (u + 2) % 8, pi)
                        row_start((u + 2) % 8, (r + 2) % 4, pi)

                    @pl.when(j + 4 < NCH)
                    def _(j=j, u=u, pi=pi):
                        idx_start(j + 4, (u + 4) % 8, pi)
                return carry

            lax.fori_loop(0, NCH // 8, body, 0)
            sc_wait(0)
            sc_wait(1)
            plsc.subcore_barrier()
            pltpu.sync_copy(acc_sh.at[pl.ds(sid * rpt, rpt)],
                            out_hbm.at[cid, p, pl.ds(sid * rpt, rpt)])

    return k(tbl_p, idx_packed, zeros)


def _tc_prep(x, W, nm_col, pd):
    def body(x_ref, w_ref, nm_ref, pd_ref, y_ref, dinv_ref):
        nm = nm_ref[...]
        deg = nm * (1.0 + pd_ref[0, 0, :, 0:1] + pd_ref[1, 0, :, 0:1])
        dinv = jnp.where(deg > 0, lax.rsqrt(jnp.maximum(deg, 1e-12)), 0.0)
        xw = jnp.dot(x_ref[...], w_ref[...], preferred_element_type=F32)
        y_ref[...] = xw * dinv
        dinv_ref[...] = dinv

    return pl.pallas_call(
        body,
        grid=(GRID,),
        in_specs=[
            pl.BlockSpec((RB, D), lambda i: (i, 0)),
            pl.BlockSpec((D, D), lambda i: (0, 0)),
            pl.BlockSpec((RB, 1), lambda i: (i, 0)),
            pl.BlockSpec((NC, 1, RB, 16), lambda i: (0, 0, i, 0)),
        ],
        out_specs=[
            pl.BlockSpec((RB, D), lambda i: (i, 0)),
            pl.BlockSpec((RB, 1), lambda i: (i, 0)),
        ],
        out_shape=[
            jax.ShapeDtypeStruct((NP, D), F32),
            jax.ShapeDtypeStruct((NP, 1), F32),
        ],
    )(x, W, nm_col, pd)


def _tc_post(Sp, y, dinv, nm_col, b2, pw2):
    def body(sp_ref, y_ref, dinv_ref, nm_ref, b_ref, pw_ref, h_ref, sc_ref):
        S = jnp.concatenate([sp_ref[0, 0] + sp_ref[1, 0],
                             sp_ref[0, 1] + sp_ref[1, 1]], axis=1)
        agg = dinv_ref[...] * (S + y_ref[...])
        h = jnp.maximum((agg + b_ref[...]) * nm_ref[...], 0.0)
        pw = pw_ref[...]
        nrm = jnp.sqrt(jnp.sum(pw * pw))
        h_ref[...] = h
        sc_ref[...] = jnp.dot(h, pw, preferred_element_type=F32) / nrm

    return pl.pallas_call(
        body,
        grid=(GRID,),
        in_specs=[
            pl.BlockSpec((NC, 2, RB, D // 2), lambda i: (0, 0, i, 0)),
            pl.BlockSpec((RB, D), lambda i: (i, 0)),
            pl.BlockSpec((RB, 1), lambda i: (i, 0)),
            pl.BlockSpec((RB, 1), lambda i: (i, 0)),
            pl.BlockSpec((1, D), lambda i: (0, 0)),
            pl.BlockSpec((D, 1), lambda i: (0, 0)),
        ],
        out_specs=[
            pl.BlockSpec((RB, D), lambda i: (i, 0)),
            pl.BlockSpec((RB, 1), lambda i: (i, 0)),
        ],
        out_shape=[
            jax.ShapeDtypeStruct((NP, D), F32),
            jax.ShapeDtypeStruct((NP, 1), F32),
        ],
    )(Sp, y, dinv, nm_col, b2, pw2)


def _tc_counts(b_row, nm_row):
    def body(br_ref, nmr_ref, kk_ref):
        gids = lax.broadcasted_iota(jnp.int32, (G, 1), 0)
        ohg = jnp.where(br_ref[...] == gids,
                        jnp.broadcast_to(nmr_ref[...], (G, NP)), 0.0)
        kk_ref[...] = jnp.ceil(0.5 * jnp.sum(ohg, axis=1, keepdims=True))

    return pl.pallas_call(
        body,
        in_specs=[
            pl.BlockSpec((1, NP), lambda: (0, 0)),
            pl.BlockSpec((1, NP), lambda: (0, 0)),
        ],
        out_specs=pl.BlockSpec((G, 1), lambda: (0, 0)),
        out_shape=jax.ShapeDtypeStruct((G, 1), F32),
    )(b_row, nm_row)


def _tc_topk(sc_col, sc_row, b_col, b_row, nm_col, nm_row, kk):
    def body(scc_ref, scr_ref, bc_ref, br_ref, nmc_ref, nmr_ref, kk_ref,
             out_ref, rep_ref, acc_ref):
        i0 = pl.program_id(0) * BI
        s_i = scc_ref[...]
        b_i = bc_ref[...]
        nm_i = nmc_ref[...]
        s_jf = scr_ref[...]
        b_jf = br_ref[...]
        nm_jf = nmr_ref[...]
        gid2 = lax.broadcasted_iota(jnp.int32, (BI, G), 1)
        ohi = (b_i == gid2).astype(F32)
        k_i = jnp.dot(ohi, kk_ref[...], preferred_element_type=F32)
        g_lo = jnp.min(b_i)
        g_hi = jnp.max(b_i)
        acc_ref[...] = jnp.zeros((BI, 1), F32)
        for c in range(NP // CJ):
            s_j = s_jf[:, c * CJ:(c + 1) * CJ]
            b_j = b_jf[:, c * CJ:(c + 1) * CJ]
            nm_j = nm_jf[:, c * CJ:(c + 1) * CJ]
            c_lo = jnp.min(b_j)
            c_hi = jnp.max(b_j)

            @pl.when(jnp.logical_not((c_hi < g_lo) | (c_lo > g_hi)))
            def _(c=c, s_j=s_j, b_j=b_j, nm_j=nm_j):
                same = b_j == b_i
                gt = s_j > s_i
                eq = s_j == s_i
                jj = lax.broadcasted_iota(jnp.int32, (BI, CJ), 1) + c * CJ
                ii = lax.broadcasted_iota(jnp.int32, (BI, CJ), 0) + i0
                bef = gt | (eq & (jj < ii))
                contrib = jnp.where(same & bef,
                                    jnp.broadcast_to(nm_j, (BI, CJ)), 0.0)
                acc_ref[...] += jnp.sum(contrib, axis=1, keepdims=True)

        rank = acc_ref[...]
        keep = jnp.where((nm_i > 0) & (rank < k_i), 1.0, 0.0)
        out_ref[...] = keep
        rep_ref[...] = jnp.broadcast_to(keep, (BI, 16))

    return pl.pallas_call(
        body,
        grid=(NP // BI,),
        in_specs=[
            pl.BlockSpec((BI, 1), lambda i: (i, 0)),
            pl.BlockSpec((1, NP), lambda i: (0, 0)),
            pl.BlockSpec((BI, 1), lambda i: (i, 0)),
            pl.BlockSpec((1, NP), lambda i: (0, 0)),
            pl.BlockSpec((BI, 1), lambda i: (i, 0)),
            pl.BlockSpec((1, NP), lambda i: (0, 0)),
            pl.BlockSpec((G, 1), lambda i: (0, 0)),
        ],
        out_specs=[
            pl.BlockSpec((BI, 1), lambda i: (i, 0)),
            pl.BlockSpec((BI, 16), lambda i: (i, 0)),
        ],
        out_shape=[
            jax.ShapeDtypeStruct((NP, 1), F32),
            jax.ShapeDtypeStruct((NP, 16), F32),
        ],
        scratch_shapes=[pltpu.VMEM((BI, 1), F32)],
    )(sc_col, sc_row, b_col, b_row, nm_col, nm_row, kk)


def _tc_pool_readout(h, sc_col, nmn_col, nmn_row, b_col, b_row):
    def body(h_ref, scc_ref, nmc_ref, nmr_ref, bc_ref, br_ref,
             hn_ref, sum_ref, cnt_ref, mx_ref):
        i = pl.program_id(0)
        nm_c = nmc_ref[...]
        hn = h_ref[...] * jnp.tanh(scc_ref[...]) * nm_c
        hn_ref[...] = hn
        gids = lax.broadcasted_iota(jnp.int32, (G, RB), 0)
        oh = jnp.where(br_ref[...] == gids,
                       jnp.broadcast_to(nmr_ref[...], (G, RB)), 0.0)

        @pl.when(i == 0)
        def _():
            sum_ref[...] = jnp.zeros((G, D), F32)
            cnt_ref[...] = jnp.zeros((G, 1), F32)
            mx_ref[...] = jnp.full((G, D), -jnp.inf, F32)

        sum_ref[...] += jnp.dot(oh, hn, preferred_element_type=F32)
        cnt_ref[...] += jnp.sum(oh, axis=1, keepdims=True)
        b_c = bc_ref[...]
        b_lo = jnp.min(b_c)
        b_hi = jnp.max(b_c)
        for g in range(G):
            @pl.when((g >= b_lo) & (g <= b_hi))
            def _(g=g):
                mcol = (b_c == g) & (nm_c > 0)
                m = jnp.where(mcol, hn, -jnp.inf)
                mx_ref[g, :] = jnp.maximum(mx_ref[g, :], jnp.max(m, axis=0))

    return pl.pallas_call(
        body,
        grid=(GRID,),
        in_specs=[
            pl.BlockSpec((RB, D), lambda i: (i, 0)),
            pl.BlockSpec((RB, 1), lambda i: (i, 0)),
            pl.BlockSpec((RB, 1), lambda i: (i, 0)),
            pl.BlockSpec((1, RB), lambda i: (0, i)),
            pl.BlockSpec((RB, 1), lambda i: (i, 0)),
            pl.BlockSpec((1, RB), lambda i: (0, i)),
        ],
        out_specs=[
            pl.BlockSpec((RB, D), lambda i: (i, 0)),
            pl.BlockSpec((G, D), lambda i: (0, 0)),
            pl.BlockSpec((G, 1), lambda i: (0, 0)),
            pl.BlockSpec((G, D), lambda i: (0, 0)),
        ],
        out_shape=[
            jax.ShapeDtypeStruct((NP, D), F32),
            jax.ShapeDtypeStruct((G, D), F32),
            jax.ShapeDtypeStruct((G, 1), F32),
            jax.ShapeDtypeStruct((G, D), F32),
        ],
    )(h, sc_col, nmn_col, nmn_row, b_col, b_row)


def _tc_head(s1, c1, m1, s2, c2, m2, w1, b1, w2, b2, w3, b3):
    def body(s1_ref, c1_ref, m1_ref, s2_ref, c2_ref, m2_ref,
             w1_ref, b1_ref, w2_ref, b2_ref, w3_ref, b3_ref, out_ref):
        def ro(s_ref, c_ref, m_ref):
            c = c_ref[...]
            mean = s_ref[...] / jnp.maximum(c, 1.0)
            mx = jnp.where(c > 0, m_ref[...], 0.0)
            return jnp.concatenate([mx, mean], axis=1)

        o = ro(s1_ref, c1_ref, m1_ref) + ro(s2_ref, c2_ref, m2_ref)
        z = jnp.maximum(jnp.dot(o, w1_ref[...], preferred_element_type=F32)
                        + b1_ref[...], 0.0)
        z = jnp.maximum(jnp.dot(z, w2_ref[...], preferred_element_type=F32)
                        + b2_ref[...], 0.0)
        z = jnp.dot(z, w3_ref[...], preferred_element_type=F32) + b3_ref[...]
        zm = jnp.max(z, axis=1, keepdims=True)
        lse = zm + jnp.log(jnp.sum(jnp.exp(z - zm), axis=1, keepdims=True))
        out_ref[...] = z - lse

    args = (s1, c1, m1, s2, c2, m2, w1, b1, w2, b2, w3, b3)
    return pl.pallas_call(
        body,
        in_specs=[pl.BlockSpec(a.shape, lambda: tuple(0 for _ in a.shape))
                  for a in args],
        out_specs=pl.BlockSpec((G, 2), lambda: (0, 0)),
        out_shape=jax.ShapeDtypeStruct((G, 2), F32),
    )(*args)


def kernel(x, edge_index, batch, W_in, b_in, pw_in, W_h, b_h, pw_h,
           lin1_w, lin1_b, lin2_w, lin2_b, lin3_w, lin3_b):
    src = edge_index[0]
    dst = edge_index[1]
    srcp = jnp.concatenate(
        [src, jnp.zeros((E_PAD - E,), jnp.int32)]).reshape(NW, NCH, CHUNK)
    dstp = jnp.concatenate(
        [dst, jnp.full((E_PAD - E,), PAD_DST, jnp.int32)]).reshape(NW, NCH, CHUNK)
    idxp = jnp.stack([srcp, dstp], axis=2).reshape(TOT_CH, 2, CHUNK)
    zeros_d = jnp.zeros((NT, D // 2), F32)
    zeros_m = jnp.zeros((NT, 16), F32)
    x_p = jnp.concatenate([x, jnp.zeros((NP - N, D), F32)], axis=0)
    nm0 = jnp.concatenate([jnp.ones((N, 1), F32),
                           jnp.zeros((NP - N, 1), F32)], axis=0)
    rep0 = jnp.broadcast_to(nm0, (NP, 16))
    b_col = jnp.concatenate(
        [batch, jnp.full((NP - N,), G - 1, jnp.int32)]).reshape(NP, 1)
    b_row = b_col.reshape(1, NP)

    def layer(x_cur, W, b2, pw2, nm_col, nm_rep):
        pd = _sc_segment_sum(nm_rep.reshape(1, NP, 16), idxp, zeros_m, 16, 1)
        y, dinv = _tc_prep(x_cur, W, nm_col, pd)
        y_pair = y.reshape(NP, 2, D // 2).transpose(1, 0, 2)
        Sp = _sc_segment_sum(y_pair, idxp, zeros_d, D // 2, 2)
        h, score = _tc_post(Sp, y, dinv, nm_col, b2, pw2)
        kk = _tc_counts(b_row, nm_col.reshape(1, NP))
        newm, rep_new = _tc_topk(score, score.reshape(1, NP), b_col, b_row,
                                 nm_col, nm_col.reshape(1, NP), kk)
        h_new, s_g, c_g, m_g = _tc_pool_readout(
            h, score, newm, newm.reshape(1, NP), b_col, b_row)
        return h_new, newm, rep_new, (s_g, c_g, m_g)

    h1, nm1, rep1, ro1 = layer(x_p, W_in, b_in.reshape(1, D),
                               pw_in.reshape(D, 1), nm0, rep0)
    _, _, _, ro2 = layer(h1, W_h, b_h.reshape(1, D),
                         pw_h.reshape(D, 1), nm1, rep1)
    return _tc_head(*ro1, *ro2, lin1_w, lin1_b.reshape(1, D),
                    lin2_w, lin2_b.reshape(1, D // 2),
                    lin3_w, lin3_b.reshape(1, 2))


# speedup vs baseline: 1.3026x; 1.3026x over previous
import functools

import jax
import jax.numpy as jnp
from jax import lax
from jax.experimental import pallas as pl
from jax.experimental.pallas import tpu as pltpu
from jax.experimental.pallas import tpu_sc as plsc

N = 10000
E = 320000
D = 128
G = 64
NC, NS = 2, 16
NW = NC * NS
CHUNK = 128
EPW = 10240
E_PAD = NW * EPW
NCH = EPW // CHUNK
TOT_CH = NW * NCH
NP = 10240
NT = NP
PAD_DST = 10100
RB = 1024
GRID = NP // RB
BI = 512
CJ = 1024
F32 = jnp.float32


def _sc_segment_sum(tbl_p, idx_packed, zeros, d2, npass):
    rpt = NT // NS
    spt = NP // NS
    mesh = plsc.VectorSubcoreMesh(core_axis_name="c", subcore_axis_name="s")

    @functools.partial(
        pl.kernel,
        out_type=jax.ShapeDtypeStruct((NC, npass, NT, d2), F32),
        mesh=mesh,
        scratch_types=[
            pltpu.VMEM((8, 2, CHUNK), jnp.int32),
            pltpu.VMEM((4, CHUNK, d2), F32),
            pltpu.VMEM_SHARED((NP, d2), F32),
            pltpu.VMEM_SHARED((NT, d2), F32),
            pltpu.SemaphoreType.DMA,
            pltpu.SemaphoreType.DMA,
            pltpu.SemaphoreType.DMA,
            pltpu.SemaphoreType.DMA,
            pltpu.SemaphoreType.DMA,
            pltpu.SemaphoreType.DMA,
        ],
        compiler_params=pltpu.CompilerParams(use_tc_tiling_on_sc=False),
    )
    def k(tbl_hbm, idx_hbm, zeros_hbm, out_hbm, idx_v, rows_v, tbl_sh,
          acc_sh, si0, si1, si2, si3, sr0, sr1):
        cid = lax.axis_index("c")
        sid = lax.axis_index("s")
        wid = sid * NC + cid
        base = wid * NCH
        sis = (si0, si1)
        srs = (si2, si3)
        sss = (sr0, sr1)

        def idx_start(c, s8, pi):
            pltpu.async_copy(idx_hbm.at[base + c], idx_v.at[s8], sis[pi])

        def idx_wait(c, s8, pi):
            pltpu.make_async_copy(idx_hbm.at[base + c], idx_v.at[s8],
                                  sis[pi]).wait()

        def row_start(s8, r, pi):
            pltpu.async_copy(tbl_sh.at[idx_v.at[s8, 0]], rows_v.at[r],
                             srs[pi])

        def row_wait(r, pi):
            pltpu.make_async_copy(tbl_sh.at[idx_v.at[0, 0]], rows_v.at[r],
                                  srs[pi]).wait()

        def sc_wait(pi):
            pltpu.make_async_copy(rows_v.at[0], acc_sh.at[idx_v.at[0, 1]],
                                  sss[pi]).wait()

        for p in range(npass):
            pltpu.sync_copy(tbl_hbm.at[p, pl.ds(sid * spt, spt)],
                            tbl_sh.at[pl.ds(sid * spt, spt)])
            pltpu.sync_copy(zeros_hbm.at[pl.ds(sid * rpt, rpt)],
                            acc_sh.at[pl.ds(sid * rpt, rpt)])
            plsc.subcore_barrier()
            idx_start(0, 0, 0)
            idx_start(1, 1, 1)
            idx_wait(0, 0, 0)
            row_start(0, 0, 0)
            idx_wait(1, 1, 1)
            row_start(1, 1, 1)
            idx_start(2, 2, 0)
            idx_start(3, 3, 1)

            def body(q, carry):
                for u in range(8):
                    j = q * 8 + u
                    pi = u % 2
                    r = u % 4
                    row_wait(r, pi)
                    if u >= 2:
                        sc_wait(pi)
                    else:
                        @pl.when(q > 0)
                        def _(pi=pi):
                            sc_wait(pi)
                    pltpu.async_copy(rows_v.at[r], acc_sh.at[idx_v.at[u, 1]],
                                     sss[pi], add=True)

                    @pl.when(j + 2 < NCH)
                    def _(j=j, u=u, r=r, pi=pi):
                        idx_wait(j + 2, (u + 2) % 8, pi)
                        row_start((u + 2) % 8, (r + 2) % 4, pi)

                    @pl.when(j + 4 < NCH)
                    def _(j=j, u=u, pi=pi):
                        idx_start(j + 4, (u + 4) % 8, pi)
                return carry

            lax.fori_loop(0, NCH // 8, body, 0)
            sc_wait(0)
            sc_wait(1)
            plsc.subcore_barrier()
            pltpu.sync_copy(acc_sh.at[pl.ds(sid * rpt, rpt)],
                            out_hbm.at[cid, p, pl.ds(sid * rpt, rpt)])

    return k(tbl_p, idx_packed, zeros)


def _tc_prep(x, W, nm_col, pd):
    def body(x_ref, w_ref, nm_ref, pd_ref, y_ref, dinv_ref):
        nm = nm_ref[...]
        deg = nm * (1.0 + pd_ref[0, 0, :, 0:1] + pd_ref[1, 0, :, 0:1])
        dinv = jnp.where(deg > 0, lax.rsqrt(jnp.maximum(deg, 1e-12)), 0.0)
        xw = jnp.dot(x_ref[...], w_ref[...], preferred_element_type=F32)
        y_ref[...] = xw * dinv
        dinv_ref[...] = dinv

    return pl.pallas_call(
        body,
        grid=(GRID,),
        in_specs=[
            pl.BlockSpec((RB, D), lambda i: (i, 0)),
            pl.BlockSpec((D, D), lambda i: (0, 0)),
            pl.BlockSpec((RB, 1), lambda i: (i, 0)),
            pl.BlockSpec((NC, 1, RB, 16), lambda i: (0, 0, i, 0)),
        ],
        out_specs=[
            pl.BlockSpec((RB, D), lambda i: (i, 0)),
            pl.BlockSpec((RB, 1), lambda i: (i, 0)),
        ],
        out_shape=[
            jax.ShapeDtypeStruct((NP, D), F32),
            jax.ShapeDtypeStruct((NP, 1), F32),
        ],
    )(x, W, nm_col, pd)


def _tc_post(Sp, y, dinv, nm_col, b2, pw2):
    def body(sp_ref, y_ref, dinv_ref, nm_ref, b_ref, pw_ref, h_ref, sc_ref):
        S = jnp.concatenate([sp_ref[0, 0] + sp_ref[1, 0],
                             sp_ref[0, 1] + sp_ref[1, 1]], axis=1)
        agg = dinv_ref[...] * (S + y_ref[...])
        h = jnp.maximum((agg + b_ref[...]) * nm_ref[...], 0.0)
        pw = pw_ref[...]
        nrm = jnp.sqrt(jnp.sum(pw * pw))
        h_ref[...] = h
        sc_ref[...] = jnp.dot(h, pw, preferred_element_type=F32) / nrm

    return pl.pallas_call(
        body,
        grid=(GRID,),
        in_specs=[
            pl.BlockSpec((NC, 2, RB, D // 2), lambda i: (0, 0, i, 0)),
            pl.BlockSpec((RB, D), lambda i: (i, 0)),
            pl.BlockSpec((RB, 1), lambda i: (i, 0)),
            pl.BlockSpec((RB, 1), lambda i: (i, 0)),
            pl.BlockSpec((1, D), lambda i: (0, 0)),
            pl.BlockSpec((D, 1), lambda i: (0, 0)),
        ],
        out_specs=[
            pl.BlockSpec((RB, D), lambda i: (i, 0)),
            pl.BlockSpec((RB, 1), lambda i: (i, 0)),
        ],
        out_shape=[
            jax.ShapeDtypeStruct((NP, D), F32),
            jax.ShapeDtypeStruct((NP, 1), F32),
        ],
    )(Sp, y, dinv, nm_col, b2, pw2)


def _tc_counts(b_row, nm_row):
    def body(br_ref, nmr_ref, kk_ref):
        gids = lax.broadcasted_iota(jnp.int32, (G, 1), 0)
        ohg = jnp.where(br_ref[...] == gids,
                        jnp.broadcast_to(nmr_ref[...], (G, NP)), 0.0)
        kk_ref[...] = jnp.ceil(0.5 * jnp.sum(ohg, axis=1, keepdims=True))

    return pl.pallas_call(
        body,
        in_specs=[
            pl.BlockSpec((1, NP), lambda: (0, 0)),
            pl.BlockSpec((1, NP), lambda: (0, 0)),
        ],
        out_specs=pl.BlockSpec((G, 1), lambda: (0, 0)),
        out_shape=jax.ShapeDtypeStruct((G, 1), F32),
    )(b_row, nm_row)


def _tc_topk(sc_col, sc_row, b_col, b_row, nm_col, nm_row, kk):
    def body(scc_ref, scr_ref, bc_ref, br_ref, nmc_ref, nmr_ref, kk_ref,
             out_ref, rep_ref, acc_ref):
        i0 = pl.program_id(0) * BI
        s_i = scc_ref[...]
        b_i = bc_ref[...]
        nm_i = nmc_ref[...]
        s_jf = scr_ref[...]
        b_jf = br_ref[...]
        nm_jf = nmr_ref[...]
        gid2 = lax.broadcasted_iota(jnp.int32, (BI, G), 1)
        ohi = (b_i == gid2).astype(F32)
        k_i = jnp.dot(ohi, kk_ref[...], preferred_element_type=F32)
        g_lo = jnp.min(b_i)
        g_hi = jnp.max(b_i)
        acc_ref[...] = jnp.zeros((BI, 1), F32)
        for c in range(NP // CJ):
            s_j = s_jf[:, c * CJ:(c + 1) * CJ]
            b_j = b_jf[:, c * CJ:(c + 1) * CJ]
            nm_j = nm_jf[:, c * CJ:(c + 1) * CJ]
            c_lo = jnp.min(b_j)
            c_hi = jnp.max(b_j)

            @pl.when(jnp.logical_not((c_hi < g_lo) | (c_lo > g_hi)))
            def _(c=c, s_j=s_j, b_j=b_j, nm_j=nm_j):
                same = b_j == b_i
                gt = s_j > s_i
                eq = s_j == s_i
                jj = lax.broadcasted_iota(jnp.int32, (BI, CJ), 1) + c * CJ
                ii = lax.broadcasted_iota(jnp.int32, (BI, CJ), 0) + i0
                bef = gt | (eq & (jj < ii))
                contrib = jnp.where(same & bef,
                                    jnp.broadcast_to(nm_j, (BI, CJ)), 0.0)
                acc_ref[...] += jnp.sum(contrib, axis=1, keepdims=True)

        rank = acc_ref[...]
        keep = jnp.where((nm_i > 0) & (rank < k_i), 1.0, 0.0)
        out_ref[...] = keep
        rep_ref[...] = jnp.broadcast_to(keep, (BI, 16))

    return pl.pallas_call(
        body,
        grid=(NP // BI,),
        in_specs=[
            pl.BlockSpec((BI, 1), lambda i: (i, 0)),
            pl.BlockSpec((1, NP), lambda i: (0, 0)),
            pl.BlockSpec((BI, 1), lambda i: (i, 0)),
            pl.BlockSpec((1, NP), lambda i: (0, 0)),
            pl.BlockSpec((BI, 1), lambda i: (i, 0)),
            pl.BlockSpec((1, NP), lambda i: (0, 0)),
            pl.BlockSpec((G, 1), lambda i: (0, 0)),
        ],
        out_specs=[
            pl.BlockSpec((BI, 1), lambda i: (i, 0)),
            pl.BlockSpec((BI, 16), lambda i: (i, 0)),
        ],
        out_shape=[
            jax.ShapeDtypeStruct((NP, 1), F32),
            jax.ShapeDtypeStruct((NP, 16), F32),
        ],
        scratch_shapes=[pltpu.VMEM((BI, 1), F32)],
    )(sc_col, sc_row, b_col, b_row, nm_col, nm_row, kk)


def _tc_pool_readout(h, sc_col, nmn_col, nmn_row, b_col, b_row):
    def body(h_ref, scc_ref, nmc_ref, nmr_ref, bc_ref, br_ref,
             hn_ref, sum_ref, cnt_ref, mx_ref):
        i = pl.program_id(0)
        nm_c = nmc_ref[...]
        hn = h_ref[...] * jnp.tanh(scc_ref[...]) * nm_c
        hn_ref[...] = hn
        gids = lax.broadcasted_iota(jnp.int32, (G, RB), 0)
        oh = jnp.where(br_ref[...] == gids,
                       jnp.broadcast_to(nmr_ref[...], (G, RB)), 0.0)

        @pl.when(i == 0)
        def _():
            sum_ref[...] = jnp.zeros((G, D), F32)
            cnt_ref[...] = jnp.zeros((G, 1), F32)
            mx_ref[...] = jnp.full((G, D), -jnp.inf, F32)

        sum_ref[...] += jnp.dot(oh, hn, preferred_element_type=F32)
        cnt_ref[...] += jnp.sum(oh, axis=1, keepdims=True)
        b_c = bc_ref[...]
        b_lo = jnp.min(b_c)
        b_hi = jnp.max(b_c)
        for g in range(G):
            @pl.when((g >= b_lo) & (g <= b_hi))
            def _(g=g):
                mcol = (b_c == g) & (nm_c > 0)
                m = jnp.where(mcol, hn, -jnp.inf)
                mx_ref[g, :] = jnp.maximum(mx_ref[g, :], jnp.max(m, axis=0))

    return pl.pallas_call(
        body,
        grid=(GRID,),
        in_specs=[
            pl.BlockSpec((RB, D), lambda i: (i, 0)),
            pl.BlockSpec((RB, 1), lambda i: (i, 0)),
            pl.BlockSpec((RB, 1), lambda i: (i, 0)),
            pl.BlockSpec((1, RB), lambda i: (0, i)),
            pl.BlockSpec((RB, 1), lambda i: (i, 0)),
            pl.BlockSpec((1, RB), lambda i: (0, i)),
        ],
        out_specs=[
            pl.BlockSpec((RB, D), lambda i: (i, 0)),
            pl.BlockSpec((G, D), lambda i: (0, 0)),
            pl.BlockSpec((G, 1), lambda i: (0, 0)),
            pl.BlockSpec((G, D), lambda i: (0, 0)),
        ],
        out_shape=[
            jax.ShapeDtypeStruct((NP, D), F32),
            jax.ShapeDtypeStruct((G, D), F32),
            jax.ShapeDtypeStruct((G, 1), F32),
            jax.ShapeDtypeStruct((G, D), F32),
        ],
    )(h, sc_col, nmn_col, nmn_row, b_col, b_row)


def _tc_head(s1, c1, m1, s2, c2, m2, w1, b1, w2, b2, w3, b3):
    def body(s1_ref, c1_ref, m1_ref, s2_ref, c2_ref, m2_ref,
             w1_ref, b1_ref, w2_ref, b2_ref, w3_ref, b3_ref, out_ref):
        def ro(s_ref, c_ref, m_ref):
            c = c_ref[...]
            mean = s_ref[...] / jnp.maximum(c, 1.0)
            mx = jnp.where(c > 0, m_ref[...], 0.0)
            return jnp.concatenate([mx, mean], axis=1)

        o = ro(s1_ref, c1_ref, m1_ref) + ro(s2_ref, c2_ref, m2_ref)
        z = jnp.maximum(jnp.dot(o, w1_ref[...], preferred_element_type=F32)
                        + b1_ref[...], 0.0)
        z = jnp.maximum(jnp.dot(z, w2_ref[...], preferred_element_type=F32)
                        + b2_ref[...], 0.0)
        z = jnp.dot(z, w3_ref[...], preferred_element_type=F32) + b3_ref[...]
        zm = jnp.max(z, axis=1, keepdims=True)
        lse = zm + jnp.log(jnp.sum(jnp.exp(z - zm), axis=1, keepdims=True))
        out_ref[...] = z - lse

    args = (s1, c1, m1, s2, c2, m2, w1, b1, w2, b2, w3, b3)
    return pl.pallas_call(
        body,
        in_specs=[pl.BlockSpec(a.shape, lambda: tuple(0 for _ in a.shape))
                  for a in args],
        out_specs=pl.BlockSpec((G, 2), lambda: (0, 0)),
        out_shape=jax.ShapeDtypeStruct((G, 2), F32),
    )(*args)


def kernel(x, edge_index, batch, W_in, b_in, pw_in, W_h, b_h, pw_h,
           lin1_w, lin1_b, lin2_w, lin2_b, lin3_w, lin3_b):
    src = edge_index[0]
    dst = edge_index[1]
    srcp = jnp.concatenate(
        [src, jnp.zeros((E_PAD - E,), jnp.int32)]).reshape(NW, NCH, CHUNK)
    dstp = jnp.concatenate(
        [dst, jnp.full((E_PAD - E,), PAD_DST, jnp.int32)]).reshape(NW, NCH, CHUNK)
    idxp = jnp.stack([srcp, dstp], axis=2).reshape(TOT_CH, 2, CHUNK)
    zeros_d = jnp.zeros((NT, D // 2), F32)
    zeros_m = jnp.zeros((NT, 16), F32)
    x_p = jnp.concatenate([x, jnp.zeros((NP - N, D), F32)], axis=0)
    nm0 = jnp.concatenate([jnp.ones((N, 1), F32),
                           jnp.zeros((NP - N, 1), F32)], axis=0)
    rep0 = jnp.broadcast_to(nm0, (NP, 16))
    b_col = jnp.concatenate(
        [batch, jnp.full((NP - N,), G - 1, jnp.int32)]).reshape(NP, 1)
    b_row = b_col.reshape(1, NP)

    def layer(x_cur, W, b2, pw2, nm_col, nm_rep):
        pd = _sc_segment_sum(nm_rep.reshape(1, NP, 16), idxp, zeros_m, 16, 1)
        y, dinv = _tc_prep(x_cur, W, nm_col, pd)
        y_pair = y.reshape(NP, 2, D // 2).transpose(1, 0, 2)
        Sp = _sc_segment_sum(y_pair, idxp, zeros_d, D // 2, 2)
        h, score = _tc_post(Sp, y, dinv, nm_col, b2, pw2)
        kk = _tc_counts(b_row, nm_col.reshape(1, NP))
        newm, rep_new = _tc_topk(score, score.reshape(1, NP), b_col, b_row,
                                 nm_col, nm_col.reshape(1, NP), kk)
        h_new, s_g, c_g, m_g = _tc_pool_readout(
            h, score, newm, newm.reshape(1, NP), b_col, b_row)
        return h_new, newm, rep_new, (s_g, c_g, m_g)

    h1, nm1, rep1, ro1 = layer(x_p, W_in, b_in.reshape(1, D),
                               pw_in.reshape(D, 1), nm0, rep0)
    _, _, _, ro2 = layer(h1, W_h, b_h.reshape(1, D),
                         pw_h.reshape(D, 1), nm1, rep1)
    return _tc_head(*ro1, *ro2, lin1_w, lin1_b.reshape(1, D),
                    lin2_w, lin2_b.reshape(1, D // 2),
                    lin3_w, lin3_b.reshape(1, 2))


# speedup vs baseline: 1.3157x; 1.0100x over previous
import functools

import jax
import jax.numpy as jnp
from jax import lax
from jax.experimental import pallas as pl
from jax.experimental.pallas import tpu as pltpu
from jax.experimental.pallas import tpu_sc as plsc

N = 10000
E = 320000
D = 128
G = 64
NC, NS = 2, 16
NW = NC * NS
CHUNK = 128
EPW = 10240
E_PAD = NW * EPW
NCH = EPW // CHUNK
TOT_CH = NW * NCH
NP = 10240
NT = NP
PAD_DST = 10100
RB = 1024
GRID = NP // RB
BI = 1024
CJ = 1024
F32 = jnp.float32


def _sc_segment_sum(tbl_p, idx_packed, zeros, d2, npass):
    rpt = NT // NS
    spt = NP // NS
    mesh = plsc.VectorSubcoreMesh(core_axis_name="c", subcore_axis_name="s")

    @functools.partial(
        pl.kernel,
        out_type=jax.ShapeDtypeStruct((NC, npass, NT, d2), F32),
        mesh=mesh,
        scratch_types=[
            pltpu.VMEM((8, 2, CHUNK), jnp.int32),
            pltpu.VMEM((4, CHUNK, d2), F32),
            pltpu.VMEM_SHARED((NP, d2), F32),
            pltpu.VMEM_SHARED((NT, d2), F32),
            pltpu.SemaphoreType.DMA,
            pltpu.SemaphoreType.DMA,
            pltpu.SemaphoreType.DMA,
            pltpu.SemaphoreType.DMA,
            pltpu.SemaphoreType.DMA,
            pltpu.SemaphoreType.DMA,
        ],
        compiler_params=pltpu.CompilerParams(use_tc_tiling_on_sc=False),
    )
    def k(tbl_hbm, idx_hbm, zeros_hbm, out_hbm, idx_v, rows_v, tbl_sh,
          acc_sh, si0, si1, si2, si3, sr0, sr1):
        cid = lax.axis_index("c")
        sid = lax.axis_index("s")
        wid = sid * NC + cid
        base = wid * NCH
        sis = (si0, si1)
        srs = (si2, si3)
        sss = (sr0, sr1)

        def idx_start(c, s8, pi):
            pltpu.async_copy(idx_hbm.at[base + c], idx_v.at[s8], sis[pi])

        def idx_wait(c, s8, pi):
            pltpu.make_async_copy(idx_hbm.at[base + c], idx_v.at[s8],
                                  sis[pi]).wait()

        def row_start(s8, r, pi):
            pltpu.async_copy(tbl_sh.at[idx_v.at[s8, 0]], rows_v.at[r],
                             srs[pi])

        def row_wait(r, pi):
            pltpu.make_async_copy(tbl_sh.at[idx_v.at[0, 0]], rows_v.at[r],
                                  srs[pi]).wait()

        def sc_wait(pi):
            pltpu.make_async_copy(rows_v.at[0], acc_sh.at[idx_v.at[0, 1]],
                                  sss[pi]).wait()

        for p in range(npass):
            pltpu.sync_copy(tbl_hbm.at[p, pl.ds(sid * spt, spt)],
                            tbl_sh.at[pl.ds(sid * spt, spt)])
            pltpu.sync_copy(zeros_hbm.at[pl.ds(sid * rpt, rpt)],
                            acc_sh.at[pl.ds(sid * rpt, rpt)])
            plsc.subcore_barrier()
            idx_start(0, 0, 0)
            idx_start(1, 1, 1)
            idx_wait(0, 0, 0)
            row_start(0, 0, 0)
            idx_wait(1, 1, 1)
            row_start(1, 1, 1)
            idx_start(2, 2, 0)
            idx_start(3, 3, 1)

            def body(q, carry):
                for u in range(8):
                    j = q * 8 + u
                    pi = u % 2
                    r = u % 4
                    row_wait(r, pi)
                    if u >= 2:
                        sc_wait(pi)
                    else:
                        @pl.when(q > 0)
                        def _(pi=pi):
                            sc_wait(pi)
                    pltpu.async_copy(rows_v.at[r], acc_sh.at[idx_v.at[u, 1]],
                                     sss[pi], add=True)

                    @pl.when(j + 2 < NCH)
                    def _(j=j, u=u, r=r, pi=pi):
                        idx_wait(j + 2, (u + 2) % 8, pi)
                        row_start((u + 2) % 8, (r + 2) % 4, pi)

                    @pl.when(j + 4 < NCH)
                    def _(j=j, u=u, pi=pi):
                        idx_start(j + 4, (u + 4) % 8, pi)
                return carry

            lax.fori_loop(0, NCH // 8, body, 0)
            sc_wait(0)
            sc_wait(1)
            plsc.subcore_barrier()
            pltpu.sync_copy(acc_sh.at[pl.ds(sid * rpt, rpt)],
                            out_hbm.at[cid, p, pl.ds(sid * rpt, rpt)])

    return k(tbl_p, idx_packed, zeros)


def _tc_prep(x, W, nm_col, pd):
    def body(x_ref, w_ref, nm_ref, pd_ref, y_ref, dinv_ref):
        nm = nm_ref[...]
        deg = nm * (1.0 + pd_ref[0, 0, :, 0:1] + pd_ref[1, 0, :, 0:1])
        dinv = jnp.where(deg > 0, lax.rsqrt(jnp.maximum(deg, 1e-12)), 0.0)
        xw = jnp.dot(x_ref[...], w_ref[...], preferred_element_type=F32)
        y_ref[...] = xw * dinv
        dinv_ref[...] = dinv

    return pl.pallas_call(
        body,
        grid=(GRID,),
        in_specs=[
            pl.BlockSpec((RB, D), lambda i: (i, 0)),
            pl.BlockSpec((D, D), lambda i: (0, 0)),
            pl.BlockSpec((RB, 1), lambda i: (i, 0)),
            pl.BlockSpec((NC, 1, RB, 16), lambda i: (0, 0, i, 0)),
        ],
        out_specs=[
            pl.BlockSpec((RB, D), lambda i: (i, 0)),
            pl.BlockSpec((RB, 1), lambda i: (i, 0)),
        ],
        out_shape=[
            jax.ShapeDtypeStruct((NP, D), F32),
            jax.ShapeDtypeStruct((NP, 1), F32),
        ],
    )(x, W, nm_col, pd)


def _tc_post(Sp, y, dinv, nm_col, b2, pw2):
    def body(sp_ref, y_ref, dinv_ref, nm_ref, b_ref, pw_ref, h_ref, sc_ref):
        S = jnp.concatenate([sp_ref[0, 0] + sp_ref[1, 0],
                             sp_ref[0, 1] + sp_ref[1, 1]], axis=1)
        agg = dinv_ref[...] * (S + y_ref[...])
        h = jnp.maximum((agg + b_ref[...]) * nm_ref[...], 0.0)
        pw = pw_ref[...]
        nrm = jnp.sqrt(jnp.sum(pw * pw))
        h_ref[...] = h
        sc_ref[...] = jnp.dot(h, pw, preferred_element_type=F32) / nrm

    return pl.pallas_call(
        body,
        grid=(GRID,),
        in_specs=[
            pl.BlockSpec((NC, 2, RB, D // 2), lambda i: (0, 0, i, 0)),
            pl.BlockSpec((RB, D), lambda i: (i, 0)),
            pl.BlockSpec((RB, 1), lambda i: (i, 0)),
            pl.BlockSpec((RB, 1), lambda i: (i, 0)),
            pl.BlockSpec((1, D), lambda i: (0, 0)),
            pl.BlockSpec((D, 1), lambda i: (0, 0)),
        ],
        out_specs=[
            pl.BlockSpec((RB, D), lambda i: (i, 0)),
            pl.BlockSpec((RB, 1), lambda i: (i, 0)),
        ],
        out_shape=[
            jax.ShapeDtypeStruct((NP, D), F32),
            jax.ShapeDtypeStruct((NP, 1), F32),
        ],
    )(Sp, y, dinv, nm_col, b2, pw2)


def _tc_counts(b_row, nm_row):
    def body(br_ref, nmr_ref, kk_ref):
        gids = lax.broadcasted_iota(jnp.int32, (G, 1), 0)
        ohg = jnp.where(br_ref[...] == gids,
                        jnp.broadcast_to(nmr_ref[...], (G, NP)), 0.0)
        kk_ref[...] = jnp.ceil(0.5 * jnp.sum(ohg, axis=1, keepdims=True))

    return pl.pallas_call(
        body,
        in_specs=[
            pl.BlockSpec((1, NP), lambda: (0, 0)),
            pl.BlockSpec((1, NP), lambda: (0, 0)),
        ],
        out_specs=pl.BlockSpec((G, 1), lambda: (0, 0)),
        out_shape=jax.ShapeDtypeStruct((G, 1), F32),
    )(b_row, nm_row)


def _tc_topk(sc_col, sc_row, b_col, b_row, nm_col, nm_row, kk):
    def body(scc_ref, scr_ref, bc_ref, br_ref, nmc_ref, nmr_ref, kk_ref,
             out_ref, rep_ref, acc_ref):
        i0 = pl.program_id(0) * BI
        s_i = scc_ref[...]
        b_i = bc_ref[...]
        nm_i = nmc_ref[...]
        s_jf = scr_ref[...]
        b_jf = br_ref[...]
        nm_jf = nmr_ref[...]
        gid2 = lax.broadcasted_iota(jnp.int32, (BI, G), 1)
        ohi = (b_i == gid2).astype(F32)
        k_i = jnp.dot(ohi, kk_ref[...], preferred_element_type=F32)
        g_lo = jnp.min(b_i)
        g_hi = jnp.max(b_i)
        acc_ref[...] = jnp.zeros((BI, 1), F32)
        for c in range(NP // CJ):
            s_j = s_jf[:, c * CJ:(c + 1) * CJ]
            b_j = b_jf[:, c * CJ:(c + 1) * CJ]
            nm_j = nm_jf[:, c * CJ:(c + 1) * CJ]
            c_lo = jnp.min(b_j)
            c_hi = jnp.max(b_j)

            @pl.when(jnp.logical_not((c_hi < g_lo) | (c_lo > g_hi)))
            def _(c=c, s_j=s_j, b_j=b_j, nm_j=nm_j):
                same = b_j == b_i
                gt = s_j > s_i
                eq = s_j == s_i
                jj = lax.broadcasted_iota(jnp.int32, (BI, CJ), 1) + c * CJ
                ii = lax.broadcasted_iota(jnp.int32, (BI, CJ), 0) + i0
                bef = gt | (eq & (jj < ii))
                contrib = jnp.where(same & bef,
                                    jnp.broadcast_to(nm_j, (BI, CJ)), 0.0)
                acc_ref[...] += jnp.sum(contrib, axis=1, keepdims=True)

        rank = acc_ref[...]
        keep = jnp.where((nm_i > 0) & (rank < k_i), 1.0, 0.0)
        out_ref[...] = keep
        rep_ref[...] = jnp.broadcast_to(keep, (BI, 16))

    return pl.pallas_call(
        body,
        grid=(NP // BI,),
        in_specs=[
            pl.BlockSpec((BI, 1), lambda i: (i, 0)),
            pl.BlockSpec((1, NP), lambda i: (0, 0)),
            pl.BlockSpec((BI, 1), lambda i: (i, 0)),
            pl.BlockSpec((1, NP), lambda i: (0, 0)),
            pl.BlockSpec((BI, 1), lambda i: (i, 0)),
            pl.BlockSpec((1, NP), lambda i: (0, 0)),
            pl.BlockSpec((G, 1), lambda i: (0, 0)),
        ],
        out_specs=[
            pl.BlockSpec((BI, 1), lambda i: (i, 0)),
            pl.BlockSpec((BI, 16), lambda i: (i, 0)),
        ],
        out_shape=[
            jax.ShapeDtypeStruct((NP, 1), F32),
            jax.ShapeDtypeStruct((NP, 16), F32),
        ],
        scratch_shapes=[pltpu.VMEM((BI, 1), F32)],
    )(sc_col, sc_row, b_col, b_row, nm_col, nm_row, kk)


def _tc_pool_readout(h, sc_col, nmn_col, nmn_row, b_col, b_row):
    def body(h_ref, scc_ref, nmc_ref, nmr_ref, bc_ref, br_ref,
             hn_ref, sum_ref, cnt_ref, mx_ref):
        i = pl.program_id(0)
        nm_c = nmc_ref[...]
        hn = h_ref[...] * jnp.tanh(scc_ref[...]) * nm_c
        hn_ref[...] = hn
        gids = lax.broadcasted_iota(jnp.int32, (G, RB), 0)
        oh = jnp.where(br_ref[...] == gids,
                       jnp.broadcast_to(nmr_ref[...], (G, RB)), 0.0)

        @pl.when(i == 0)
        def _():
            sum_ref[...] = jnp.zeros((G, D), F32)
            cnt_ref[...] = jnp.zeros((G, 1), F32)
            mx_ref[...] = jnp.full((G, D), -jnp.inf, F32)

        sum_ref[...] += jnp.dot(oh, hn, preferred_element_type=F32)
        cnt_ref[...] += jnp.sum(oh, axis=1, keepdims=True)
        b_c = bc_ref[...]
        b_lo = jnp.min(b_c)
        b_hi = jnp.max(b_c)
        for g in range(G):
            @pl.when((g >= b_lo) & (g <= b_hi))
            def _(g=g):
                mcol = (b_c == g) & (nm_c > 0)
                m = jnp.where(mcol, hn, -jnp.inf)
                mx_ref[g, :] = jnp.maximum(mx_ref[g, :], jnp.max(m, axis=0))

    return pl.pallas_call(
        body,
        grid=(GRID,),
        in_specs=[
            pl.BlockSpec((RB, D), lambda i: (i, 0)),
            pl.BlockSpec((RB, 1), lambda i: (i, 0)),
            pl.BlockSpec((RB, 1), lambda i: (i, 0)),
            pl.BlockSpec((1, RB), lambda i: (0, i)),
            pl.BlockSpec((RB, 1), lambda i: (i, 0)),
            pl.BlockSpec((1, RB), lambda i: (0, i)),
        ],
        out_specs=[
            pl.BlockSpec((RB, D), lambda i: (i, 0)),
            pl.BlockSpec((G, D), lambda i: (0, 0)),
            pl.BlockSpec((G, 1), lambda i: (0, 0)),
            pl.BlockSpec((G, D), lambda i: (0, 0)),
        ],
        out_shape=[
            jax.ShapeDtypeStruct((NP, D), F32),
            jax.ShapeDtypeStruct((G, D), F32),
            jax.ShapeDtypeStruct((G, 1), F32),
            jax.ShapeDtypeStruct((G, D), F32),
        ],
    )(h, sc_col, nmn_col, nmn_row, b_col, b_row)


def _tc_head(s1, c1, m1, s2, c2, m2, w1, b1, w2, b2, w3, b3):
    def body(s1_ref, c1_ref, m1_ref, s2_ref, c2_ref, m2_ref,
             w1_ref, b1_ref, w2_ref, b2_ref, w3_ref, b3_ref, out_ref):
        def ro(s_ref, c_ref, m_ref):
            c = c_ref[...]
            mean = s_ref[...] / jnp.maximum(c, 1.0)
            mx = jnp.where(c > 0, m_ref[...], 0.0)
            return jnp.concatenate([mx, mean], axis=1)

        o = ro(s1_ref, c1_ref, m1_ref) + ro(s2_ref, c2_ref, m2_ref)
        z = jnp.maximum(jnp.dot(o, w1_ref[...], preferred_element_type=F32)
                        + b1_ref[...], 0.0)
        z = jnp.maximum(jnp.dot(z, w2_ref[...], preferred_element_type=F32)
                        + b2_ref[...], 0.0)
        z = jnp.dot(z, w3_ref[...], preferred_element_type=F32) + b3_ref[...]
        zm = jnp.max(z, axis=1, keepdims=True)
        lse = zm + jnp.log(jnp.sum(jnp.exp(z - zm), axis=1, keepdims=True))
        out_ref[...] = z - lse

    args = (s1, c1, m1, s2, c2, m2, w1, b1, w2, b2, w3, b3)
    return pl.pallas_call(
        body,
        in_specs=[pl.BlockSpec(a.shape, lambda: tuple(0 for _ in a.shape))
                  for a in args],
        out_specs=pl.BlockSpec((G, 2), lambda: (0, 0)),
        out_shape=jax.ShapeDtypeStruct((G, 2), F32),
    )(*args)


def kernel(x, edge_index, batch, W_in, b_in, pw_in, W_h, b_h, pw_h,
           lin1_w, lin1_b, lin2_w, lin2_b, lin3_w, lin3_b):
    src = edge_index[0]
    dst = edge_index[1]
    srcp = jnp.concatenate(
        [src, jnp.zeros((E_PAD - E,), jnp.int32)]).reshape(NW, NCH, CHUNK)
    dstp = jnp.concatenate(
        [dst, jnp.full((E_PAD - E,), PAD_DST, jnp.int32)]).reshape(NW, NCH, CHUNK)
    idxp = jnp.stack([srcp, dstp], axis=2).reshape(TOT_CH, 2, CHUNK)
    zeros_d = jnp.zeros((NT, D // 2), F32)
    zeros_m = jnp.zeros((NT, 16), F32)
    x_p = jnp.concatenate([x, jnp.zeros((NP - N, D), F32)], axis=0)
    nm0 = jnp.concatenate([jnp.ones((N, 1), F32),
                           jnp.zeros((NP - N, 1), F32)], axis=0)
    rep0 = jnp.broadcast_to(nm0, (NP, 16))
    b_col = jnp.concatenate(
        [batch, jnp.full((NP - N,), G - 1, jnp.int32)]).reshape(NP, 1)
    b_row = b_col.reshape(1, NP)

    def layer(x_cur, W, b2, pw2, nm_col, nm_rep):
        pd = _sc_segment_sum(nm_rep.reshape(1, NP, 16), idxp, zeros_m, 16, 1)
        y, dinv = _tc_prep(x_cur, W, nm_col, pd)
        y_pair = y.reshape(NP, 2, D // 2).transpose(1, 0, 2)
        Sp = _sc_segment_sum(y_pair, idxp, zeros_d, D // 2, 2)
        h, score = _tc_post(Sp, y, dinv, nm_col, b2, pw2)
        kk = _tc_counts(b_row, nm_col.reshape(1, NP))
        newm, rep_new = _tc_topk(score, score.reshape(1, NP), b_col, b_row,
                                 nm_col, nm_col.reshape(1, NP), kk)
        h_new, s_g, c_g, m_g = _tc_pool_readout(
            h, score, newm, newm.reshape(1, NP), b_col, b_row)
        return h_new, newm, rep_new, (s_g, c_g, m_g)

    h1, nm1, rep1, ro1 = layer(x_p, W_in, b_in.reshape(1, D),
                               pw_in.reshape(D, 1), nm0, rep0)
    _, _, _, ro2 = layer(h1, W_h, b_h.reshape(1, D),
                         pw_h.reshape(D, 1), nm1, rep1)
    return _tc_head(*ro1, *ro2, lin1_w, lin1_b.reshape(1, D),
                    lin2_w, lin2_b.reshape(1, D // 2),
                    lin3_w, lin3_b.reshape(1, 2))
